# baseline (speedup 1.0000x reference)
import functools

import numpy as np

import jax
import jax.numpy as jnp
from jax import lax
from jax.experimental import pallas as pl
from jax.experimental.pallas import tpu as pltpu
from jax.experimental.pallas import tpu_sc as plsc

N = 10000
NPAD = 10240
D_FEAT = 128
HID = 16
NCLS = 16
E = 320000
K = 128
EB = E // K
NC, NS = 2, 16
NW = NC * NS
TPB = 82
IB = NW * TPB - EB
R = NPAD // NS

f32 = jnp.float32
i32 = jnp.int32


def _tc1_body(x_ref, w_ref, asrc_ref, adst_ref, h_ref, as_ref, ad_ref):
    h = x_ref[...] @ w_ref[...]
    h_ref[...] = h
    as_ref[pl.ds(0, N)] = jnp.sum(h * asrc_ref[...], axis=1)
    ad_ref[pl.ds(0, N)] = jnp.sum(h * adst_ref[...], axis=1)


_tc1 = pl.pallas_call(
    _tc1_body,
    out_shape=[
        jax.ShapeDtypeStruct((N, HID), f32),
        jax.ShapeDtypeStruct((NPAD,), f32),
        jax.ShapeDtypeStruct((NPAD,), f32),
    ],
)


def _tc2_body(num_ref, den_ref, b_ref, w_ref, asrc_ref, adst_ref,
              h_ref, as_ref, ad_ref):
    num = num_ref[pl.ds(0, N), :] + num_ref[pl.ds(NPAD, N), :]
    den = den_ref[0, pl.ds(0, N)] + den_ref[1, pl.ds(0, N)]
    xo = num / (den[:, None] + 1e-16) + b_ref[...]
    xo = jnp.maximum(xo, 0.0)
    h = xo @ w_ref[...]
    h_ref[...] = h
    as_ref[pl.ds(0, N)] = jnp.sum(h * asrc_ref[...], axis=1)
    ad_ref[pl.ds(0, N)] = jnp.sum(h * adst_ref[...], axis=1)


_tc2 = pl.pallas_call(
    _tc2_body,
    out_shape=[
        jax.ShapeDtypeStruct((N, HID), f32),
        jax.ShapeDtypeStruct((NPAD,), f32),
        jax.ShapeDtypeStruct((NPAD,), f32),
    ],
)


def _tc3_body(num_ref, den_ref, b_ref, w_ref, bo_ref, out_ref):
    num = num_ref[pl.ds(0, N), :] + num_ref[pl.ds(NPAD, N), :]
    den = den_ref[0, pl.ds(0, N)] + den_ref[1, pl.ds(0, N)]
    xo = num / (den[:, None] + 1e-16) + b_ref[...]
    xo = jnp.maximum(xo, 0.0)
    logits = xo @ w_ref[...] + bo_ref[...]
    m = jnp.max(logits, axis=1, keepdims=True)
    p = jnp.exp(logits - m)
    out_ref[...] = p / jnp.sum(p, axis=1, keepdims=True)


_tc3 = pl.pallas_call(
    _tc3_body,
    out_shape=jax.ShapeDtypeStruct((N, NCLS), f32),
)


_mesh = plsc.VectorSubcoreMesh(core_axis_name="c", subcore_axis_name="s")


@functools.partial(
    pl.kernel,
    out_type=(
        jax.ShapeDtypeStruct((NC * NPAD, HID), f32),
        jax.ShapeDtypeStruct((NC, NPAD), f32),
    ),
    mesh=_mesh,
    compiler_params=pltpu.CompilerParams(needs_layout_passes=False,
                                         use_tc_tiling_on_sc=False),
    scratch_types=[
        pltpu.VMEM((NPAD,), f32),
        pltpu.VMEM((NPAD,), f32),
        pltpu.VMEM((NPAD,), f32),
        pltpu.VMEM((TPB, K), i32),
        pltpu.VMEM((TPB, K), i32),
        pltpu.VMEM((K,), f32),
        pltpu.VMEM((K, HID), f32),
        pltpu.VMEM((K, HID), f32),
        pltpu.VMEM((NS, R), f32),
        pltpu.VMEM_SHARED((NPAD, HID), f32),
        pltpu.VMEM_SHARED((NS, NS, R), f32),
        pltpu.SemaphoreType.DMA,
        pltpu.SemaphoreType.DMA,
        pltpu.SemaphoreType.DMA,
        pltpu.SemaphoreType.DMA,
    ],
)
def _sc_edge(h_hbm, as_hbm, ad_hbm, se_hbm, de_hbm, is_hbm, id_hbm,
             num_out, den_out,
             as_l, ad_l, den_l, src_l, dst_l, exb, rows0, rows1, dtmp,
             num_sh, den_sh, sem_g0, sem_g1, sem_s0, sem_s1):
    c = lax.axis_index("c")
    s = lax.axis_index("s")
    wid = c * NS + s

    pltpu.sync_copy(as_hbm, as_l)
    pltpu.sync_copy(ad_hbm, ad_l)

    @pl.when(wid < 4)
    def _():
        pltpu.sync_copy(se_hbm.at[pl.ds(79 * wid, 79)], src_l.at[pl.ds(0, 79)])
        pltpu.sync_copy(de_hbm.at[pl.ds(79 * wid, 79)], dst_l.at[pl.ds(0, 79)])
        pltpu.sync_copy(is_hbm.at[pl.ds(3 * wid, 3)], src_l.at[pl.ds(79, 3)])
        pltpu.sync_copy(id_hbm.at[pl.ds(3 * wid, 3)], dst_l.at[pl.ds(79, 3)])

    @pl.when(wid >= 4)
    def _():
        pltpu.sync_copy(se_hbm.at[pl.ds(78 * wid + 4, 78)],
                        src_l.at[pl.ds(0, 78)])
        pltpu.sync_copy(de_hbm.at[pl.ds(78 * wid + 4, 78)],
                        dst_l.at[pl.ds(0, 78)])
        pltpu.sync_copy(is_hbm.at[pl.ds(4 * wid - 4, 4)],
                        src_l.at[pl.ds(78, 4)])
        pltpu.sync_copy(id_hbm.at[pl.ds(4 * wid - 4, 4)],
                        dst_l.at[pl.ds(78, 4)])

    zeros16 = jnp.zeros((16,), f32)

    def _zero_den(j, carry):
        den_l[pl.ds(j * 16, 16)] = zeros16
        return carry

    lax.fori_loop(0, NPAD // 16, _zero_den, 0)

    def _zero_rows(j, carry):
        rows0[j] = zeros16
        return carry

    lax.fori_loop(0, K, _zero_rows, 0)

    def _zero_num(i, carry):
        pltpu.sync_copy(rows0, num_sh.at[pl.ds(s * R + i * K, K)])
        return carry

    lax.fori_loop(0, R // K, _zero_num, 0)
    plsc.subcore_barrier()

    def _phase1(b):
        def _grp(g, carry2):
            sl = pl.ds(g * 16, 16)
            s16 = src_l[b, sl]
            d16 = dst_l[b, sl]
            e = plsc.load_gather(as_l, [s16]) + plsc.load_gather(ad_l, [d16])
            e = jnp.maximum(e, e * 0.2)
            ex = jnp.exp(e)
            ex = jnp.where(d16 < N, ex, 0.0)
            plsc.addupdate_scatter(den_l, [d16], ex)
            exb[sl] = ex
            return carry2

        lax.fori_loop(0, K // 16, _grp, 0)

    def _scale(buf):
        def _s(g, carry2):
            exv = exb[pl.ds(g * 16, 16)]
            for j in range(16):
                r = g * 16 + j
                buf[r] = buf[r] * exv[j]
            return carry2

        lax.fori_loop(0, K // 16, _s, 0)

    S = TPB // 2
    pltpu.async_copy(h_hbm.at[src_l.at[0]], rows0, sem_g0)

    def _super(i, carry):
        b0 = 2 * i
        b1 = b0 + 1
        _phase1(b0)

        @pl.when(i > 0)
        def _():
            pltpu.make_async_copy(
                rows1, num_sh.at[dst_l.at[b0 - 1]], sem_s1).wait()

        pltpu.async_copy(h_hbm.at[src_l.at[b1]], rows1, sem_g1)
        pltpu.make_async_copy(h_hbm.at[src_l.at[b0]], rows0, sem_g0).wait()
        _scale(rows0)
        pltpu.async_copy(rows0, num_sh.at[dst_l.at[b0]], sem_s0, add=True)
        _phase1(b1)

        @pl.when(i < S - 1)
        def _():
            pltpu.make_async_copy(
                rows0, num_sh.at[dst_l.at[b0]], sem_s0).wait()
            pltpu.async_copy(h_hbm.at[src_l.at[b0 + 2]], rows0, sem_g0)

        pltpu.make_async_copy(h_hbm.at[src_l.at[b1]], rows1, sem_g1).wait()
        _scale(rows1)
        pltpu.async_copy(rows1, num_sh.at[dst_l.at[b1]], sem_s1, add=True)
        return carry

    lax.fori_loop(0, S, _super, 0)
    pltpu.make_async_copy(rows0, num_sh.at[dst_l.at[TPB - 2]], sem_s0).wait()
    pltpu.make_async_copy(rows1, num_sh.at[dst_l.at[TPB - 1]], sem_s1).wait()

    def _wden(j, carry):
        pltpu.sync_copy(den_l.at[pl.ds(j * R, R)], den_sh.at[j, s])
        return carry

    lax.fori_loop(0, NS, _wden, 0)
    plsc.subcore_barrier()

    pltpu.sync_copy(den_sh.at[s], dtmp)

    def _dred(g, carry):
        sl = pl.ds(g * 16, 16)
        acc = dtmp[0, sl]
        for t in range(1, NS):
            acc = acc + dtmp[t, sl]
        den_l[sl] = acc
        return carry

    lax.fori_loop(0, R // 16, _dred, 0)

    pltpu.sync_copy(den_l.at[pl.ds(0, R)], den_out.at[c, pl.ds(s * R, R)])
    pltpu.sync_copy(num_sh.at[pl.ds(s * R, R)],
                    num_out.at[pl.ds(c * NPAD + s * R, R)])


_idn_vals = np.concatenate(
    [np.arange(NPAD, dtype=np.int32),
     np.full((IB * K - NPAD,), N, dtype=np.int32)]).reshape(IB, K)
_idn_src = np.minimum(_idn_vals, N - 1)


def kernel(x, edge_index, W1, att_src1, att_dst1, b1,
           W2, att_src2, att_dst2, b2, Wout, bout):
    se2d = edge_index[0].reshape(EB, K)
    de2d = edge_index[1].reshape(EB, K)

    h1, as1, ad1 = _tc1(x, W1, att_src1.reshape(1, HID),
                        att_dst1.reshape(1, HID))
    num1, den1 = _sc_edge(h1, as1, ad1, se2d, de2d, _idn_src, _idn_vals)
    h2, as2, ad2 = _tc2(num1, den1, b1.reshape(1, HID), W2,
                        att_src2.reshape(1, HID), att_dst2.reshape(1, HID))
    num2, den2 = _sc_edge(h2, as2, ad2, se2d, de2d, _idn_src, _idn_vals)
    return _tc3(num2, den2, b2.reshape(1, HID), Wout, bout.reshape(1, NCLS))


# speedup vs baseline: 1.1038x; 1.1038x over previous
import functools

import numpy as np

import jax
import jax.numpy as jnp
from jax import lax
from jax.experimental import pallas as pl
from jax.experimental.pallas import tpu as pltpu
from jax.experimental.pallas import tpu_sc as plsc

N = 10000
NPAD = 10240
D_FEAT = 128
HID = 16
NCLS = 16
E = 320000
K = 128
EB = E // K
NC, NS = 2, 16
NW = NC * NS
TPB = 82
IB = NW * TPB - EB
R = NPAD // NS

f32 = jnp.float32
i32 = jnp.int32


def _tc1_body(x_ref, w_ref, asrc_ref, adst_ref, h_ref, as_ref, ad_ref):
    h = x_ref[...] @ w_ref[...]
    h_ref[...] = h
    as_ref[pl.ds(0, N)] = jnp.sum(h * asrc_ref[...], axis=1)
    ad_ref[pl.ds(0, N)] = jnp.sum(h * adst_ref[...], axis=1)


_tc1 = pl.pallas_call(
    _tc1_body,
    out_shape=[
        jax.ShapeDtypeStruct((N, HID), f32),
        jax.ShapeDtypeStruct((NPAD,), f32),
        jax.ShapeDtypeStruct((NPAD,), f32),
    ],
)


def _tc2_body(num_ref, den_ref, b_ref, w_ref, asrc_ref, adst_ref,
              h_ref, as_ref, ad_ref):
    num = num_ref[pl.ds(0, N), :] + num_ref[pl.ds(NPAD, N), :]
    den = den_ref[pl.ds(0, N)]
    for k in range(1, NW):
        den = den + den_ref[pl.ds(k * NPAD, N)]
    xo = num / (den[:, None] + 1e-16) + b_ref[...]
    xo = jnp.maximum(xo, 0.0)
    h = xo @ w_ref[...]
    h_ref[...] = h
    as_ref[pl.ds(0, N)] = jnp.sum(h * asrc_ref[...], axis=1)
    ad_ref[pl.ds(0, N)] = jnp.sum(h * adst_ref[...], axis=1)


_tc2 = pl.pallas_call(
    _tc2_body,
    out_shape=[
        jax.ShapeDtypeStruct((N, HID), f32),
        jax.ShapeDtypeStruct((NPAD,), f32),
        jax.ShapeDtypeStruct((NPAD,), f32),
    ],
)


def _tc3_body(num_ref, den_ref, b_ref, w_ref, bo_ref, out_ref):
    num = num_ref[pl.ds(0, N), :] + num_ref[pl.ds(NPAD, N), :]
    den = den_ref[pl.ds(0, N)]
    for k in range(1, NW):
        den = den + den_ref[pl.ds(k * NPAD, N)]
    xo = num / (den[:, None] + 1e-16) + b_ref[...]
    xo = jnp.maximum(xo, 0.0)
    logits = xo @ w_ref[...] + bo_ref[...]
    m = jnp.max(logits, axis=1, keepdims=True)
    p = jnp.exp(logits - m)
    out_ref[...] = p / jnp.sum(p, axis=1, keepdims=True)


_tc3 = pl.pallas_call(
    _tc3_body,
    out_shape=jax.ShapeDtypeStruct((N, NCLS), f32),
)


_mesh = plsc.VectorSubcoreMesh(core_axis_name="c", subcore_axis_name="s")


@functools.partial(
    pl.kernel,
    out_type=(
        jax.ShapeDtypeStruct((NC * NPAD, HID), f32),
        jax.ShapeDtypeStruct((NW * NPAD,), f32),
    ),
    mesh=_mesh,
    compiler_params=pltpu.CompilerParams(needs_layout_passes=False,
                                         use_tc_tiling_on_sc=False),
    scratch_types=[
        pltpu.VMEM((NPAD,), f32),
        pltpu.VMEM((NPAD,), f32),
        pltpu.VMEM((NPAD,), f32),
        pltpu.VMEM((TPB, K), i32),
        pltpu.VMEM((TPB, K), i32),
        pltpu.VMEM((K,), f32),
        pltpu.VMEM((K, HID), f32),
        pltpu.VMEM((K, HID), f32),
        pltpu.VMEM_SHARED((NPAD, HID), f32),
        pltpu.SemaphoreType.DMA,
        pltpu.SemaphoreType.DMA,
        pltpu.SemaphoreType.DMA,
        pltpu.SemaphoreType.DMA,
    ],
)
def _sc_edge(h_hbm, as_hbm, ad_hbm, se_hbm, de_hbm, iss_hbm, ids_hbm,
             ist_hbm, idt_hbm, num_out, den_out,
             as_l, ad_l, den_l, src_l, dst_l, exb, rows0, rows1,
             num_sh, sem_g0, sem_g1, sem_s0, sem_s1):
    c = lax.axis_index("c")
    s = lax.axis_index("s")
    wid = c * NS + s

    pltpu.sync_copy(as_hbm, as_l)
    pltpu.sync_copy(ad_hbm, ad_l)

    @pl.when(wid < NW - 1)
    def _():
        pltpu.sync_copy(se_hbm.at[pl.ds(80 * wid, 80)], src_l.at[pl.ds(0, 80)])
        pltpu.sync_copy(de_hbm.at[pl.ds(80 * wid, 80)], dst_l.at[pl.ds(0, 80)])
        pltpu.sync_copy(iss_hbm.at[wid], src_l.at[pl.ds(80, 2)])
        pltpu.sync_copy(ids_hbm.at[wid], dst_l.at[pl.ds(80, 2)])

    @pl.when(wid == NW - 1)
    def _():
        pltpu.sync_copy(se_hbm.at[pl.ds(2480, 20)], src_l.at[pl.ds(0, 20)])
        pltpu.sync_copy(de_hbm.at[pl.ds(2480, 20)], dst_l.at[pl.ds(0, 20)])
        pltpu.sync_copy(ist_hbm, src_l.at[pl.ds(20, 62)])
        pltpu.sync_copy(idt_hbm, dst_l.at[pl.ds(20, 62)])

    zeros16 = jnp.zeros((16,), f32)

    def _zero_den(j, carry):
        den_l[pl.ds(j * 16, 16)] = zeros16
        return carry

    lax.fori_loop(0, NPAD // 16, _zero_den, 0)

    def _zero_rows(j, carry):
        rows0[j] = zeros16
        return carry

    lax.fori_loop(0, K, _zero_rows, 0)

    def _zero_num(i, carry):
        pltpu.sync_copy(rows0, num_sh.at[pl.ds(s * R + i * K, K)])
        return carry

    lax.fori_loop(0, R // K, _zero_num, 0)
    plsc.subcore_barrier()

    def _phase1(b):
        def _grp(g, carry2):
            sl = pl.ds(g * 16, 16)
            s16 = src_l[b, sl]
            d16 = dst_l[b, sl]
            e = plsc.load_gather(as_l, [s16]) + plsc.load_gather(ad_l, [d16])
            e = jnp.maximum(e, e * 0.2)
            ex = jnp.exp(e)
            ex = jnp.where(d16 < N, ex, 0.0)
            plsc.addupdate_scatter(den_l, [d16], ex)
            exb[sl] = ex
            return carry2

        lax.fori_loop(0, K // 16, _grp, 0)

    def _scale(buf):
        def _s(g, carry2):
            exv = exb[pl.ds(g * 16, 16)]
            for j in range(16):
                r = g * 16 + j
                buf[r] = buf[r] * exv[j]
            return carry2

        lax.fori_loop(0, K // 16, _s, 0)

    S = TPB // 2
    pltpu.async_copy(h_hbm.at[src_l.at[0]], rows0, sem_g0)

    def _super(i, carry):
        b0 = 2 * i
        b1 = b0 + 1
        _phase1(b0)

        @pl.when(i > 0)
        def _():
            pltpu.make_async_copy(
                rows1, num_sh.at[dst_l.at[b0 - 1]], sem_s1).wait()

        pltpu.async_copy(h_hbm.at[src_l.at[b1]], rows1, sem_g1)
        pltpu.make_async_copy(h_hbm.at[src_l.at[b0]], rows0, sem_g0).wait()
        _scale(rows0)
        pltpu.async_copy(rows0, num_sh.at[dst_l.at[b0]], sem_s0, add=True)
        _phase1(b1)

        @pl.when(i < S - 1)
        def _():
            pltpu.make_async_copy(
                rows0, num_sh.at[dst_l.at[b0]], sem_s0).wait()
            pltpu.async_copy(h_hbm.at[src_l.at[b0 + 2]], rows0, sem_g0)

        pltpu.make_async_copy(h_hbm.at[src_l.at[b1]], rows1, sem_g1).wait()
        _scale(rows1)
        pltpu.async_copy(rows1, num_sh.at[dst_l.at[b1]], sem_s1, add=True)
        return carry

    lax.fori_loop(0, S, _super, 0)
    pltpu.make_async_copy(rows0, num_sh.at[dst_l.at[TPB - 2]], sem_s0).wait()
    pltpu.make_async_copy(rows1, num_sh.at[dst_l.at[TPB - 1]], sem_s1).wait()

    plsc.subcore_barrier()
    pltpu.sync_copy(den_l, den_out.at[pl.ds(wid * NPAD, NPAD)])
    pltpu.sync_copy(num_sh.at[pl.ds(s * R, R)],
                    num_out.at[pl.ds(c * NPAD + s * R, R)])


_idn_vals = np.concatenate(
    [np.arange(NPAD, dtype=np.int32),
     np.full((IB * K - NPAD,), N, dtype=np.int32)]).reshape(IB, K)
_idn_src = np.minimum(_idn_vals, N - 1)
_ids_small = _idn_src[:62].reshape(NW - 1, 2, K)
_idd_small = _idn_vals[:62].reshape(NW - 1, 2, K)
_ids_tail = _idn_src[62:]
_idd_tail = _idn_vals[62:]


def kernel(x, edge_index, W1, att_src1, att_dst1, b1,
           W2, att_src2, att_dst2, b2, Wout, bout):
    se2d = edge_index[0].reshape(EB, K)
    de2d = edge_index[1].reshape(EB, K)

    h1, as1, ad1 = _tc1(x, W1, att_src1.reshape(1, HID),
                        att_dst1.reshape(1, HID))
    num1, den1 = _sc_edge(h1, as1, ad1, se2d, de2d, _ids_small, _idd_small,
                          _ids_tail, _idd_tail)
    h2, as2, ad2 = _tc2(num1, den1, b1.reshape(1, HID), W2,
                        att_src2.reshape(1, HID), att_dst2.reshape(1, HID))
    num2, den2 = _sc_edge(h2, as2, ad2, se2d, de2d, _ids_small, _idd_small,
                          _ids_tail, _idd_tail)
    return _tc3(num2, den2, b2.reshape(1, HID), Wout, bout.reshape(1, NCLS))
